# baseline (speedup 1.0000x reference)
import jax
import jax.numpy as jnp
import numpy as np
from jax import lax
from jax.experimental import pallas as pl
from jax.experimental.pallas import tpu as pltpu
from jax.experimental.pallas import tpu_sc as plsc

_FOUR_MASKS = 8
_FOUR_STATES = 65536
_FIVE_MASKS = 8
_FIVE_STATES = 59049
_R = 996680
_D = 64
_H = 32
_B = 16384
_M = 16

_NC, _NS, _L = 2, 16, 16
_NW = _NC * _NS

_NT_FULL = _R // 128
_TAIL = _R - _NT_FULL * 128
_TAIL_WID = _NT_FULL % _NW

_BPW = _B // _NW
_CHUNK = 32
_NCHUNK = _BPW // _CHUNK
_CROWS = _CHUNK * _M
_GSPLIT = 4
_GROWS = _CROWS // _GSPLIT


def _xpose_body(tabt_hbm, tail_hbm, flat_hbm, in0, in1, out0, out1,
                isem0, isem1, osem0, osem1):
    wid = lax.axis_index("s") * _NC + lax.axis_index("c")
    ins = (in0, in1)
    outs = (out0, out1)
    isems = (isem0, isem1)
    osems = (osem0, osem1)

    iota = lax.iota(jnp.int32, _L)
    zero16 = jnp.zeros_like(iota)
    rots = [(iota + k) & (_L - 1) for k in range(_L)]
    lconsts = [iota * 128 + r for r in rots]
    sconsts = [r * _D + iota for r in rots]

    nk = (_NT_FULL - wid + _NW - 1) // _NW
    nk2 = (nk + 1) // 2

    def fire(k, buf):
        @pl.when(k < nk)
        def _():
            t = wid + k * _NW
            pltpu.async_copy(
                tabt_hbm.at[:, pl.ds(t * 128, 128)], ins[buf], isems[buf]
            )

    def compute(k, buf):
        t = wid + k * _NW
        pltpu.make_async_copy(
            tabt_hbm.at[:, pl.ds(t * 128, 128)], ins[buf], isems[buf]
        ).wait()

        @pl.when(k >= 2)
        def _():
            pltpu.make_async_copy(
                outs[buf], flat_hbm.at[pl.ds(0, 128 * _D)], osems[buf]
            ).wait()

        def block(bi, _):
            f0 = (bi % 4) * _L
            l0 = (bi // 4) * _L
            lbase = f0 * 128 + l0
            sbase = l0 * _D + f0
            for kk in range(_L):
                v = plsc.load_gather(ins[buf], [zero16, lconsts[kk] + lbase])
                plsc.store_scatter(outs[buf], [sconsts[kk] + sbase], v)
            return 0

        lax.fori_loop(0, 32, block, 0)
        pltpu.async_copy(
            outs[buf], flat_hbm.at[pl.ds(t * 128 * _D, 128 * _D)], osems[buf]
        )

    fire(0, 0)

    def k2_body(k2, _):
        k = k2 * 2
        fire(k + 1, 1)
        compute(k, 0)
        fire(k + 2, 0)

        @pl.when(k + 1 < nk)
        def _():
            compute(k + 1, 1)

        return 0

    lax.fori_loop(0, nk2, k2_body, 0)
    for b in range(2):
        pltpu.make_async_copy(
            outs[b], flat_hbm.at[pl.ds(0, 128 * _D)], osems[b]
        ).wait()

    @pl.when(wid == _TAIL_WID)
    def _():
        pltpu.sync_copy(tail_hbm, in0)

        def block(bi, _):
            f0 = (bi % 4) * _L
            l0 = (bi // 4) * _L
            lbase = f0 * 128 + l0
            sbase = l0 * _D + f0
            for kk in range(_L):
                v = plsc.load_gather(in0, [zero16, lconsts[kk] + lbase])
                plsc.store_scatter(out0, [sconsts[kk] + sbase], v)
            return 0

        lax.fori_loop(0, 4 * ((_TAIL + _L - 1) // _L), block, 0)
        pltpu.sync_copy(
            out0.at[pl.ds(0, _TAIL * _D)],
            flat_hbm.at[pl.ds(_NT_FULL * 128 * _D, _TAIL * _D)],
        )


_xpose = pl.kernel(
    _xpose_body,
    out_type=jax.ShapeDtypeStruct((_R * _D,), jnp.float32),
    mesh=plsc.VectorSubcoreMesh(
        core_axis_name="c", subcore_axis_name="s",
        num_cores=_NC, num_subcores=_NS,
    ),
    scratch_types=[
        pltpu.VMEM((_D, 128), jnp.float32),
        pltpu.VMEM((_D, 128), jnp.float32),
        pltpu.VMEM((128 * _D,), jnp.float32),
        pltpu.VMEM((128 * _D,), jnp.float32),
        pltpu.SemaphoreType.DMA,
        pltpu.SemaphoreType.DMA,
        pltpu.SemaphoreType.DMA,
        pltpu.SemaphoreType.DMA,
    ],
    compiler_params=pltpu.CompilerParams(
        use_tc_tiling_on_sc=True, needs_layout_passes=False
    ),
)


def _sc_pool_body(idx_hbm, table_hbm, acc_hbm, idx_v, rows_v, out_v, sem0, sem1):
    wid = lax.axis_index("s") * _NC + lax.axis_index("c")
    ebase = wid * _BPW
    sems = (sem0, sem1)

    lane = lax.iota(jnp.int32, _L)
    offs = jnp.where(
        lane < _FOUR_MASKS,
        lane * _FOUR_STATES,
        _FOUR_MASKS * _FOUR_STATES + (lane - _FOUR_MASKS) * _FIVE_STATES,
    )

    def fire(g, buf):
        pltpu.sync_copy(
            idx_hbm.at[pl.ds((ebase + g * _CHUNK) * _M, _CROWS)],
            idx_v.at[buf],
        )

        def fix(e, _):
            sl = pl.ds(e * _M, _M)
            idx_v[buf, sl] = idx_v[buf, sl] + offs
            return 0

        lax.fori_loop(0, _CHUNK, fix, 0)
        return [
            pltpu.async_copy(
                table_hbm.at[idx_v.at[buf, pl.ds(j * _GROWS, _GROWS)]],
                rows_v.at[buf, pl.ds(j * _GROWS, _GROWS)],
                sems[buf],
            )
            for j in range(_GSPLIT)
        ]

    def pool(g, buf):
        def elem(e, _):
            row0 = e * _M
            for q in range(_D // _L):
                cs = pl.ds(q * _L, _L)
                s = rows_v[buf, row0, cs]
                for r in range(1, _M):
                    s = s + rows_v[buf, row0 + r, cs]
                out_v[buf, e, cs] = jnp.minimum(jnp.maximum(s, 0.0), 1.0)
            return 0

        lax.fori_loop(0, _CHUNK, elem, 0)
        pltpu.sync_copy(
            out_v.at[buf],
            acc_hbm.at[pl.ds(ebase + g * _CHUNK, _CHUNK)],
        )

    pending = fire(0, 0)
    for g in range(_NCHUNK):
        buf = g & 1
        current = pending
        if g + 1 < _NCHUNK:
            pending = fire(g + 1, 1 - buf)
        for h in current:
            h.wait()
        pool(g, buf)


_sc_pool = pl.kernel(
    _sc_pool_body,
    out_type=jax.ShapeDtypeStruct((_B, _D), jnp.float32),
    mesh=plsc.VectorSubcoreMesh(
        core_axis_name="c", subcore_axis_name="s",
        num_cores=_NC, num_subcores=_NS,
    ),
    scratch_types=[
        pltpu.VMEM((2, _CROWS), jnp.int32),
        pltpu.VMEM((2, _CROWS, _D), jnp.float32),
        pltpu.VMEM((2, _CHUNK, _D), jnp.float32),
        pltpu.SemaphoreType.DMA,
        pltpu.SemaphoreType.DMA,
    ],
    compiler_params=pltpu.CompilerParams(use_tc_tiling_on_sc=False),
)


_MLP_BLK = 2048


def _mlp_body(acc_ref, w1_ref, b1_ref, w2_ref, b2_ref, out_ref):
    a = acc_ref[...]
    h = jnp.dot(a, w1_ref[...], preferred_element_type=jnp.float32) + b1_ref[...]
    h = jnp.minimum(jnp.maximum(h, 0.0), 1.0)
    o = jnp.sum(h * w2_ref[...], axis=1) + b2_ref[0, 0]
    out_ref[...] = jnp.tanh(o)


_mlp = pl.pallas_call(
    _mlp_body,
    grid=(_B // _MLP_BLK,),
    in_specs=[
        pl.BlockSpec((_MLP_BLK, _D), lambda i: (i, 0)),
        pl.BlockSpec((_D, _H), lambda i: (0, 0)),
        pl.BlockSpec((1, _H), lambda i: (0, 0)),
        pl.BlockSpec((1, _H), lambda i: (0, 0)),
        pl.BlockSpec(memory_space=pltpu.SMEM),
    ],
    out_specs=pl.BlockSpec((_MLP_BLK,), lambda i: (i,)),
    out_shape=jax.ShapeDtypeStruct((_B,), jnp.float32),
)


def kernel(four_states, five_states, table, hidden_w, hidden_b, output_w, output_b):
    idx = jnp.concatenate([four_states, five_states], axis=1).reshape(-1)
    tabt = table.T
    tail = jnp.pad(tabt[:, _NT_FULL * 128 :], ((0, 0), (0, 128 - _TAIL)))
    flat = _xpose(tabt, tail)
    acc = _sc_pool(idx, flat.reshape(_R, _D))
    return _mlp(
        acc,
        hidden_w,
        hidden_b.reshape(1, _H),
        output_w.reshape(1, _H),
        output_b.reshape(1, 1),
    )


# speedup vs baseline: 1.8729x; 1.8729x over previous
import jax
import jax.numpy as jnp
import numpy as np
from jax import lax
from jax.experimental import pallas as pl
from jax.experimental.pallas import tpu as pltpu
from jax.experimental.pallas import tpu_sc as plsc

_FOUR_MASKS = 8
_FOUR_STATES = 65536
_FIVE_MASKS = 8
_FIVE_STATES = 59049
_R = 996680
_D = 64
_H = 32
_B = 16384
_M = 16

_NC, _NS, _L = 2, 16, 16
_NW = _NC * _NS

_NT_FULL = _R // 128
_TAIL = _R - _NT_FULL * 128
_TAIL_WID = _NT_FULL % _NW

_BPW = _B // _NW
_CHUNK = 32
_NCHUNK = _BPW // _CHUNK
_CROWS = _CHUNK * _M
_GSPLIT = 4
_GROWS = _CROWS // _GSPLIT


def _xpose_body(tabt_hbm, tail_hbm, flat_hbm, in0, in1, out0, out1,
                isem0, isem1, osem0, osem1):
    wid = lax.axis_index("s") * _NC + lax.axis_index("c")
    ins = (in0, in1)
    outs = (out0, out1)
    isems = (isem0, isem1)
    osems = (osem0, osem1)

    iota = lax.iota(jnp.int32, _L)
    zero16 = jnp.zeros_like(iota)
    rots = [(iota + k) & (_L - 1) for k in range(_L)]
    lconsts = [iota * 128 + r for r in rots]
    sconsts = [r * _D + iota for r in rots]

    nk = (_NT_FULL - wid + _NW - 1) // _NW
    nk2 = (nk + 1) // 2

    def fire(k, buf):
        @pl.when(k < nk)
        def _():
            t = wid + k * _NW
            pltpu.async_copy(
                tabt_hbm.at[:, pl.ds(t * 128, 128)], ins[buf], isems[buf]
            )

    def compute(k, buf):
        t = wid + k * _NW
        pltpu.make_async_copy(
            tabt_hbm.at[:, pl.ds(t * 128, 128)], ins[buf], isems[buf]
        ).wait()

        @pl.when(k >= 2)
        def _():
            pltpu.make_async_copy(
                outs[buf], flat_hbm.at[pl.ds(0, 128 * _D)], osems[buf]
            ).wait()

        @plsc.parallel_loop(0, 32, unroll=2)
        def block(bi):
            f0 = (bi % 4) * _L
            l0 = (bi // 4) * _L
            lbase = f0 * 128 + l0
            sbase = l0 * _D + f0
            for kk in range(_L):
                v = plsc.load_gather(ins[buf], [zero16, lconsts[kk] + lbase])
                plsc.store_scatter(outs[buf], [sconsts[kk] + sbase], v)

        pltpu.async_copy(
            outs[buf], flat_hbm.at[pl.ds(t * 128 * _D, 128 * _D)], osems[buf]
        )

    fire(0, 0)

    def k2_body(k2, _):
        k = k2 * 2
        fire(k + 1, 1)
        compute(k, 0)
        fire(k + 2, 0)

        @pl.when(k + 1 < nk)
        def _():
            compute(k + 1, 1)

        return 0

    lax.fori_loop(0, nk2, k2_body, 0)
    for b in range(2):
        pltpu.make_async_copy(
            outs[b], flat_hbm.at[pl.ds(0, 128 * _D)], osems[b]
        ).wait()

    @pl.when(wid == _TAIL_WID)
    def _():
        pltpu.sync_copy(tail_hbm, in0)

        @plsc.parallel_loop(0, 4 * ((_TAIL + _L - 1) // _L))
        def tail_block(bi):
            f0 = (bi % 4) * _L
            l0 = (bi // 4) * _L
            lbase = f0 * 128 + l0
            sbase = l0 * _D + f0
            for kk in range(_L):
                v = plsc.load_gather(in0, [zero16, lconsts[kk] + lbase])
                plsc.store_scatter(out0, [sconsts[kk] + sbase], v)

        pltpu.sync_copy(
            out0.at[pl.ds(0, _TAIL * _D)],
            flat_hbm.at[pl.ds(_NT_FULL * 128 * _D, _TAIL * _D)],
        )


_xpose = pl.kernel(
    _xpose_body,
    out_type=jax.ShapeDtypeStruct((_R * _D,), jnp.float32),
    mesh=plsc.VectorSubcoreMesh(
        core_axis_name="c", subcore_axis_name="s",
        num_cores=_NC, num_subcores=_NS,
    ),
    scratch_types=[
        pltpu.VMEM((_D, 128), jnp.float32),
        pltpu.VMEM((_D, 128), jnp.float32),
        pltpu.VMEM((128 * _D,), jnp.float32),
        pltpu.VMEM((128 * _D,), jnp.float32),
        pltpu.SemaphoreType.DMA,
        pltpu.SemaphoreType.DMA,
        pltpu.SemaphoreType.DMA,
        pltpu.SemaphoreType.DMA,
    ],
    compiler_params=pltpu.CompilerParams(
        use_tc_tiling_on_sc=True, needs_layout_passes=False
    ),
)


def _sc_pool_body(idx_hbm, table_hbm, acc_hbm, idx_v, rows_v, out_v, sem0, sem1):
    wid = lax.axis_index("s") * _NC + lax.axis_index("c")
    ebase = wid * _BPW
    sems = (sem0, sem1)

    lane = lax.iota(jnp.int32, _L)
    offs = jnp.where(
        lane < _FOUR_MASKS,
        lane * _FOUR_STATES,
        _FOUR_MASKS * _FOUR_STATES + (lane - _FOUR_MASKS) * _FIVE_STATES,
    )

    def fire(g, buf):
        pltpu.sync_copy(
            idx_hbm.at[pl.ds((ebase + g * _CHUNK) * _M, _CROWS)],
            idx_v.at[buf],
        )

        def fix(e, _):
            sl = pl.ds(e * _M, _M)
            idx_v[buf, sl] = idx_v[buf, sl] + offs
            return 0

        lax.fori_loop(0, _CHUNK, fix, 0)
        return [
            pltpu.async_copy(
                table_hbm.at[idx_v.at[buf, pl.ds(j * _GROWS, _GROWS)]],
                rows_v.at[buf, pl.ds(j * _GROWS, _GROWS)],
                sems[buf],
            )
            for j in range(_GSPLIT)
        ]

    def pool(g, buf):
        def elem(e, _):
            row0 = e * _M
            for q in range(_D // _L):
                cs = pl.ds(q * _L, _L)
                s = rows_v[buf, row0, cs]
                for r in range(1, _M):
                    s = s + rows_v[buf, row0 + r, cs]
                out_v[buf, e, cs] = jnp.minimum(jnp.maximum(s, 0.0), 1.0)
            return 0

        lax.fori_loop(0, _CHUNK, elem, 0)
        pltpu.sync_copy(
            out_v.at[buf],
            acc_hbm.at[pl.ds(ebase + g * _CHUNK, _CHUNK)],
        )

    pending = fire(0, 0)
    for g in range(_NCHUNK):
        buf = g & 1
        current = pending
        if g + 1 < _NCHUNK:
            pending = fire(g + 1, 1 - buf)
        for h in current:
            h.wait()
        pool(g, buf)


_sc_pool = pl.kernel(
    _sc_pool_body,
    out_type=jax.ShapeDtypeStruct((_B, _D), jnp.float32),
    mesh=plsc.VectorSubcoreMesh(
        core_axis_name="c", subcore_axis_name="s",
        num_cores=_NC, num_subcores=_NS,
    ),
    scratch_types=[
        pltpu.VMEM((2, _CROWS), jnp.int32),
        pltpu.VMEM((2, _CROWS, _D), jnp.float32),
        pltpu.VMEM((2, _CHUNK, _D), jnp.float32),
        pltpu.SemaphoreType.DMA,
        pltpu.SemaphoreType.DMA,
    ],
    compiler_params=pltpu.CompilerParams(use_tc_tiling_on_sc=False),
)


_MLP_BLK = 2048


def _mlp_body(acc_ref, w1_ref, b1_ref, w2_ref, b2_ref, out_ref):
    a = acc_ref[...]
    h = jnp.dot(a, w1_ref[...], preferred_element_type=jnp.float32) + b1_ref[...]
    h = jnp.minimum(jnp.maximum(h, 0.0), 1.0)
    o = jnp.sum(h * w2_ref[...], axis=1) + b2_ref[0, 0]
    out_ref[...] = jnp.tanh(o)


_mlp = pl.pallas_call(
    _mlp_body,
    grid=(_B // _MLP_BLK,),
    in_specs=[
        pl.BlockSpec((_MLP_BLK, _D), lambda i: (i, 0)),
        pl.BlockSpec((_D, _H), lambda i: (0, 0)),
        pl.BlockSpec((1, _H), lambda i: (0, 0)),
        pl.BlockSpec((1, _H), lambda i: (0, 0)),
        pl.BlockSpec(memory_space=pltpu.SMEM),
    ],
    out_specs=pl.BlockSpec((_MLP_BLK,), lambda i: (i,)),
    out_shape=jax.ShapeDtypeStruct((_B,), jnp.float32),
)


def kernel(four_states, five_states, table, hidden_w, hidden_b, output_w, output_b):
    idx = jnp.concatenate([four_states, five_states], axis=1).reshape(-1)
    tabt = table.T
    tail = jnp.pad(tabt[:, _NT_FULL * 128 :], ((0, 0), (0, 128 - _TAIL)))
    flat = _xpose(tabt, tail)
    acc = _sc_pool(idx, flat.reshape(_R, _D))
    return _mlp(
        acc,
        hidden_w,
        hidden_b.reshape(1, _H),
        output_w.reshape(1, _H),
        output_b.reshape(1, 1),
    )


# speedup vs baseline: 2.0154x; 1.0760x over previous
import jax
import jax.numpy as jnp
import numpy as np
from jax import lax
from jax.experimental import pallas as pl
from jax.experimental.pallas import tpu as pltpu
from jax.experimental.pallas import tpu_sc as plsc

_FOUR_MASKS = 8
_FOUR_STATES = 65536
_FIVE_MASKS = 8
_FIVE_STATES = 59049
_R = 996680
_D = 64
_H = 32
_B = 16384
_M = 16

_NC, _NS, _L = 2, 16, 16
_NW = _NC * _NS

_NT_FULL = _R // 128
_TAIL = _R - _NT_FULL * 128
_TAIL_WID = _NT_FULL % _NW

_BPW = _B // _NW
_CHUNK = 32
_NCHUNK = _BPW // _CHUNK
_CROWS = _CHUNK * _M
_GSPLIT = 4
_GROWS = _CROWS // _GSPLIT


def _xpose_body(tabt_hbm, tail_hbm, flat_hbm, in0, in1, out0, out1,
                isem0, isem1, osem0, osem1):
    wid = lax.axis_index("s") * _NC + lax.axis_index("c")
    ins = (in0, in1)
    outs = (out0, out1)
    isems = (isem0, isem1)
    osems = (osem0, osem1)

    iota = lax.iota(jnp.int32, _L)
    zero16 = jnp.zeros_like(iota)
    rots = [(iota + k) & (_L - 1) for k in range(_L)]
    lconsts = [iota * 128 + r for r in rots]
    sconsts = [r * _D + iota for r in rots]

    nk = (_NT_FULL - wid + _NW - 1) // _NW
    nk2 = (nk + 1) // 2

    def fire(k, buf):
        @pl.when(k < nk)
        def _():
            t = wid + k * _NW
            pltpu.async_copy(
                tabt_hbm.at[:, pl.ds(t * 128, 128)], ins[buf], isems[buf]
            )

    def compute(k, buf):
        t = wid + k * _NW
        pltpu.make_async_copy(
            tabt_hbm.at[:, pl.ds(t * 128, 128)], ins[buf], isems[buf]
        ).wait()

        @pl.when(k >= 2)
        def _():
            pltpu.make_async_copy(
                outs[buf], flat_hbm.at[pl.ds(0, 128 * _D)], osems[buf]
            ).wait()

        @plsc.parallel_loop(0, 32, unroll=4)
        def block(bi):
            f0 = (bi % 4) * _L
            l0 = (bi // 4) * _L
            lbase = f0 * 128 + l0
            sbase = l0 * _D + f0
            for kk in range(_L):
                v = plsc.load_gather(ins[buf], [zero16, lconsts[kk] + lbase])
                plsc.store_scatter(outs[buf], [sconsts[kk] + sbase], v)

        pltpu.async_copy(
            outs[buf], flat_hbm.at[pl.ds(t * 128 * _D, 128 * _D)], osems[buf]
        )

    fire(0, 0)

    def k2_body(k2, _):
        k = k2 * 2
        fire(k + 1, 1)
        compute(k, 0)
        fire(k + 2, 0)

        @pl.when(k + 1 < nk)
        def _():
            compute(k + 1, 1)

        return 0

    lax.fori_loop(0, nk2, k2_body, 0)
    for b in range(2):
        pltpu.make_async_copy(
            outs[b], flat_hbm.at[pl.ds(0, 128 * _D)], osems[b]
        ).wait()

    @pl.when(wid == _TAIL_WID)
    def _():
        pltpu.sync_copy(tail_hbm, in0)

        @plsc.parallel_loop(0, 4 * ((_TAIL + _L - 1) // _L))
        def tail_block(bi):
            f0 = (bi % 4) * _L
            l0 = (bi // 4) * _L
            lbase = f0 * 128 + l0
            sbase = l0 * _D + f0
            for kk in range(_L):
                v = plsc.load_gather(in0, [zero16, lconsts[kk] + lbase])
                plsc.store_scatter(out0, [sconsts[kk] + sbase], v)

        pltpu.sync_copy(
            out0.at[pl.ds(0, _TAIL * _D)],
            flat_hbm.at[pl.ds(_NT_FULL * 128 * _D, _TAIL * _D)],
        )


_xpose = pl.kernel(
    _xpose_body,
    out_type=jax.ShapeDtypeStruct((_R * _D,), jnp.float32),
    mesh=plsc.VectorSubcoreMesh(
        core_axis_name="c", subcore_axis_name="s",
        num_cores=_NC, num_subcores=_NS,
    ),
    scratch_types=[
        pltpu.VMEM((_D, 128), jnp.float32),
        pltpu.VMEM((_D, 128), jnp.float32),
        pltpu.VMEM((128 * _D,), jnp.float32),
        pltpu.VMEM((128 * _D,), jnp.float32),
        pltpu.SemaphoreType.DMA,
        pltpu.SemaphoreType.DMA,
        pltpu.SemaphoreType.DMA,
        pltpu.SemaphoreType.DMA,
    ],
    compiler_params=pltpu.CompilerParams(
        use_tc_tiling_on_sc=True, needs_layout_passes=False
    ),
)


def _sc_pool_body(idx_hbm, table_hbm, acc_hbm, idx_v, rows_v, out_v, sem0, sem1):
    wid = lax.axis_index("s") * _NC + lax.axis_index("c")
    ebase = wid * _BPW
    sems = (sem0, sem1)

    lane = lax.iota(jnp.int32, _L)
    offs = jnp.where(
        lane < _FOUR_MASKS,
        lane * _FOUR_STATES,
        _FOUR_MASKS * _FOUR_STATES + (lane - _FOUR_MASKS) * _FIVE_STATES,
    )

    def fire(g, buf):
        pltpu.sync_copy(
            idx_hbm.at[pl.ds((ebase + g * _CHUNK) * _M, _CROWS)],
            idx_v.at[buf],
        )

        @plsc.parallel_loop(0, _CHUNK, unroll=4)
        def fix(e):
            sl = pl.ds(e * _M, _M)
            idx_v[buf, sl] = idx_v[buf, sl] + offs

        return [
            pltpu.async_copy(
                table_hbm.at[idx_v.at[buf, pl.ds(j * _GROWS, _GROWS)]],
                rows_v.at[buf, pl.ds(j * _GROWS, _GROWS)],
                sems[buf],
            )
            for j in range(_GSPLIT)
        ]

    def pool(g, buf):
        @plsc.parallel_loop(0, _CHUNK, unroll=2)
        def elem(e):
            row0 = e * _M
            for q in range(_D // _L):
                cs = pl.ds(q * _L, _L)
                s = rows_v[buf, row0, cs]
                for r in range(1, _M):
                    s = s + rows_v[buf, row0 + r, cs]
                out_v[buf, e, cs] = jnp.minimum(jnp.maximum(s, 0.0), 1.0)

        pltpu.sync_copy(
            out_v.at[buf],
            acc_hbm.at[pl.ds(ebase + g * _CHUNK, _CHUNK)],
        )

    pending = fire(0, 0)
    for g in range(_NCHUNK):
        buf = g & 1
        current = pending
        if g + 1 < _NCHUNK:
            pending = fire(g + 1, 1 - buf)
        for h in current:
            h.wait()
        pool(g, buf)


_sc_pool = pl.kernel(
    _sc_pool_body,
    out_type=jax.ShapeDtypeStruct((_B, _D), jnp.float32),
    mesh=plsc.VectorSubcoreMesh(
        core_axis_name="c", subcore_axis_name="s",
        num_cores=_NC, num_subcores=_NS,
    ),
    scratch_types=[
        pltpu.VMEM((2, _CROWS), jnp.int32),
        pltpu.VMEM((2, _CROWS, _D), jnp.float32),
        pltpu.VMEM((2, _CHUNK, _D), jnp.float32),
        pltpu.SemaphoreType.DMA,
        pltpu.SemaphoreType.DMA,
    ],
    compiler_params=pltpu.CompilerParams(use_tc_tiling_on_sc=False),
)


_MLP_BLK = 2048


def _mlp_body(acc_ref, w1_ref, b1_ref, w2_ref, b2_ref, out_ref):
    a = acc_ref[...]
    h = jnp.dot(a, w1_ref[...], preferred_element_type=jnp.float32) + b1_ref[...]
    h = jnp.minimum(jnp.maximum(h, 0.0), 1.0)
    o = jnp.sum(h * w2_ref[...], axis=1) + b2_ref[0, 0]
    out_ref[...] = jnp.tanh(o)


_mlp = pl.pallas_call(
    _mlp_body,
    grid=(_B // _MLP_BLK,),
    in_specs=[
        pl.BlockSpec((_MLP_BLK, _D), lambda i: (i, 0)),
        pl.BlockSpec((_D, _H), lambda i: (0, 0)),
        pl.BlockSpec((1, _H), lambda i: (0, 0)),
        pl.BlockSpec((1, _H), lambda i: (0, 0)),
        pl.BlockSpec(memory_space=pltpu.SMEM),
    ],
    out_specs=pl.BlockSpec((_MLP_BLK,), lambda i: (i,)),
    out_shape=jax.ShapeDtypeStruct((_B,), jnp.float32),
)


def kernel(four_states, five_states, table, hidden_w, hidden_b, output_w, output_b):
    idx = jnp.concatenate([four_states, five_states], axis=1).reshape(-1)
    tabt = table.T
    tail = jnp.pad(tabt[:, _NT_FULL * 128 :], ((0, 0), (0, 128 - _TAIL)))
    flat = _xpose(tabt, tail)
    acc = _sc_pool(idx, flat.reshape(_R, _D))
    return _mlp(
        acc,
        hidden_w,
        hidden_b.reshape(1, _H),
        output_w.reshape(1, _H),
        output_b.reshape(1, 1),
    )


# speedup vs baseline: 2.0155x; 1.0001x over previous
import jax
import jax.numpy as jnp
import numpy as np
from jax import lax
from jax.experimental import pallas as pl
from jax.experimental.pallas import tpu as pltpu
from jax.experimental.pallas import tpu_sc as plsc

_FOUR_MASKS = 8
_FOUR_STATES = 65536
_FIVE_MASKS = 8
_FIVE_STATES = 59049
_R = 996680
_D = 64
_H = 32
_B = 16384
_M = 16

_NC, _NS, _L = 2, 16, 16
_NW = _NC * _NS

_NT_FULL = _R // 128
_TAIL = _R - _NT_FULL * 128
_PW = 128
_TAIL_WID = _NT_FULL % _NW

_BPW = _B // _NW
_CHUNK = 32
_NCHUNK = _BPW // _CHUNK
_CROWS = _CHUNK * _M
_GSPLIT = 4
_GROWS = _CROWS // _GSPLIT


def _xpose_body(tabt_hbm, tail_hbm, flat_hbm, in0, in1, out0, out1,
                isem0, isem1, osem0, osem1):
    wid = lax.axis_index("s") * _NC + lax.axis_index("c")
    ins = (in0, in1)
    outs = (out0, out1)
    isems = (isem0, isem1)
    osems = (osem0, osem1)

    iota = lax.iota(jnp.int32, _L)
    zero16 = jnp.zeros_like(iota)
    rots = [(iota + k) & (_L - 1) for k in range(_L)]
    lconsts = [iota * _PW + r for r in rots]
    sconsts = [r * _D + iota for r in rots]

    nk = (_NT_FULL - wid + _NW - 1) // _NW
    nk2 = (nk + 1) // 2

    def fire(k, buf):
        @pl.when(k < nk)
        def _():
            t = wid + k * _NW
            pltpu.async_copy(
                tabt_hbm.at[:, pl.ds(t * _PW, _PW)], ins[buf], isems[buf]
            )

    def compute(k, buf):
        t = wid + k * _NW
        pltpu.make_async_copy(
            tabt_hbm.at[:, pl.ds(t * _PW, _PW)], ins[buf], isems[buf]
        ).wait()

        @pl.when(k >= 2)
        def _():
            pltpu.make_async_copy(
                outs[buf], flat_hbm.at[pl.ds(0, _PW * _D)], osems[buf]
            ).wait()

        @plsc.parallel_loop(0, 32, unroll=4)
        def block(bi):
            f0 = (bi % 4) * _L
            l0 = (bi // 4) * _L
            lbase = f0 * _PW + l0
            sbase = l0 * _D + f0
            for kk in range(_L):
                v = plsc.load_gather(ins[buf], [zero16, lconsts[kk] + lbase])
                plsc.store_scatter(outs[buf], [sconsts[kk] + sbase], v)

        pltpu.async_copy(
            outs[buf], flat_hbm.at[pl.ds(t * _PW * _D, _PW * _D)], osems[buf]
        )

    fire(0, 0)

    def k2_body(k2, _):
        k = k2 * 2
        fire(k + 1, 1)
        compute(k, 0)
        fire(k + 2, 0)

        @pl.when(k + 1 < nk)
        def _():
            compute(k + 1, 1)

        return 0

    lax.fori_loop(0, nk2, k2_body, 0)
    for b in range(2):
        pltpu.make_async_copy(
            outs[b], flat_hbm.at[pl.ds(0, _PW * _D)], osems[b]
        ).wait()

    @pl.when(wid == _TAIL_WID)
    def _():
        pltpu.sync_copy(tail_hbm, in0)

        @plsc.parallel_loop(0, 4 * ((_TAIL + _L - 1) // _L))
        def tail_block(bi):
            f0 = (bi % 4) * _L
            l0 = (bi // 4) * _L
            lbase = f0 * _PW + l0
            sbase = l0 * _D + f0
            for kk in range(_L):
                v = plsc.load_gather(in0, [zero16, lconsts[kk] + lbase])
                plsc.store_scatter(out0, [sconsts[kk] + sbase], v)

        pltpu.sync_copy(
            out0.at[pl.ds(0, _TAIL * _D)],
            flat_hbm.at[pl.ds(_NT_FULL * 128 * _D, _TAIL * _D)],
        )


_xpose = pl.kernel(
    _xpose_body,
    out_type=jax.ShapeDtypeStruct((_R * _D,), jnp.float32),
    mesh=plsc.VectorSubcoreMesh(
        core_axis_name="c", subcore_axis_name="s",
        num_cores=_NC, num_subcores=_NS,
    ),
    scratch_types=[
        pltpu.VMEM((_D, _PW), jnp.float32),
        pltpu.VMEM((_D, _PW), jnp.float32),
        pltpu.VMEM((_PW * _D,), jnp.float32),
        pltpu.VMEM((_PW * _D,), jnp.float32),
        pltpu.SemaphoreType.DMA,
        pltpu.SemaphoreType.DMA,
        pltpu.SemaphoreType.DMA,
        pltpu.SemaphoreType.DMA,
    ],
    compiler_params=pltpu.CompilerParams(
        use_tc_tiling_on_sc=True, needs_layout_passes=False
    ),
)


def _sc_pool_body(idx_hbm, table_hbm, acc_hbm, idx_v, rows_v, out_v, sem0, sem1):
    wid = lax.axis_index("s") * _NC + lax.axis_index("c")
    ebase = wid * _BPW
    sems = (sem0, sem1)

    lane = lax.iota(jnp.int32, _L)
    offs = jnp.where(
        lane < _FOUR_MASKS,
        lane * _FOUR_STATES,
        _FOUR_MASKS * _FOUR_STATES + (lane - _FOUR_MASKS) * _FIVE_STATES,
    )

    def fire(g, buf):
        pltpu.sync_copy(
            idx_hbm.at[pl.ds((ebase + g * _CHUNK) * _M, _CROWS)],
            idx_v.at[buf],
        )

        @plsc.parallel_loop(0, _CHUNK, unroll=4)
        def fix(e):
            sl = pl.ds(e * _M, _M)
            idx_v[buf, sl] = idx_v[buf, sl] + offs

        return [
            pltpu.async_copy(
                table_hbm.at[idx_v.at[buf, pl.ds(j * _GROWS, _GROWS)]],
                rows_v.at[buf, pl.ds(j * _GROWS, _GROWS)],
                sems[buf],
            )
            for j in range(_GSPLIT)
        ]

    def pool(g, buf):
        @plsc.parallel_loop(0, _CHUNK, unroll=2)
        def elem(e):
            row0 = e * _M
            for q in range(_D // _L):
                cs = pl.ds(q * _L, _L)
                s = rows_v[buf, row0, cs]
                for r in range(1, _M):
                    s = s + rows_v[buf, row0 + r, cs]
                out_v[buf, e, cs] = jnp.minimum(jnp.maximum(s, 0.0), 1.0)

        pltpu.sync_copy(
            out_v.at[buf],
            acc_hbm.at[pl.ds(ebase + g * _CHUNK, _CHUNK)],
        )

    pending = fire(0, 0)
    for g in range(_NCHUNK):
        buf = g & 1
        current = pending
        if g + 1 < _NCHUNK:
            pending = fire(g + 1, 1 - buf)
        for h in current:
            h.wait()
        pool(g, buf)


_sc_pool = pl.kernel(
    _sc_pool_body,
    out_type=jax.ShapeDtypeStruct((_B, _D), jnp.float32),
    mesh=plsc.VectorSubcoreMesh(
        core_axis_name="c", subcore_axis_name="s",
        num_cores=_NC, num_subcores=_NS,
    ),
    scratch_types=[
        pltpu.VMEM((2, _CROWS), jnp.int32),
        pltpu.VMEM((2, _CROWS, _D), jnp.float32),
        pltpu.VMEM((2, _CHUNK, _D), jnp.float32),
        pltpu.SemaphoreType.DMA,
        pltpu.SemaphoreType.DMA,
    ],
    compiler_params=pltpu.CompilerParams(use_tc_tiling_on_sc=False),
)


_MLP_BLK = 2048


def _mlp_body(acc_ref, w1_ref, b1_ref, w2_ref, b2_ref, out_ref):
    a = acc_ref[...]
    h = jnp.dot(a, w1_ref[...], preferred_element_type=jnp.float32) + b1_ref[...]
    h = jnp.minimum(jnp.maximum(h, 0.0), 1.0)
    o = jnp.sum(h * w2_ref[...], axis=1) + b2_ref[0, 0]
    out_ref[...] = jnp.tanh(o)


_mlp = pl.pallas_call(
    _mlp_body,
    grid=(_B // _MLP_BLK,),
    in_specs=[
        pl.BlockSpec((_MLP_BLK, _D), lambda i: (i, 0)),
        pl.BlockSpec((_D, _H), lambda i: (0, 0)),
        pl.BlockSpec((1, _H), lambda i: (0, 0)),
        pl.BlockSpec((1, _H), lambda i: (0, 0)),
        pl.BlockSpec(memory_space=pltpu.SMEM),
    ],
    out_specs=pl.BlockSpec((_MLP_BLK,), lambda i: (i,)),
    out_shape=jax.ShapeDtypeStruct((_B,), jnp.float32),
)


def kernel(four_states, five_states, table, hidden_w, hidden_b, output_w, output_b):
    idx = jnp.concatenate([four_states, five_states], axis=1).reshape(-1)
    tabt = table.T
    tail = jnp.pad(tabt[:, _NT_FULL * 128 :], ((0, 0), (0, 128 - _TAIL)))
    flat = _xpose(tabt, tail)
    acc = _sc_pool(idx, flat.reshape(_R, _D))
    return _mlp(
        acc,
        hidden_w,
        hidden_b.reshape(1, _H),
        output_w.reshape(1, _H),
        output_b.reshape(1, 1),
    )
